# baseline (speedup 1.0000x reference)
import functools

import jax
import jax.numpy as jnp
from jax import lax
from jax.experimental import pallas as pl
from jax.experimental.pallas import tpu as pltpu
from jax.experimental.pallas import tpu_sc as plsc

_BN_EPS = 1e-5
_NC = 2
_NS = 16
_NW = _NC * _NS
_CHUNK = 48
_LANES = 16
_G = 24


_NBUF = 3


@functools.lru_cache(maxsize=None)
def _make_aggregate(n_pad: int, d: int, nch: int):
    assert d % _LANES == 0 and n_pad % (_NS * 8) == 0
    assert nch % _G == 0 and nch % 8 == 0 and _G % _NBUF == 0 and _G % 8 == 0
    ngr = nch // _G
    rows_per_tile = n_pad // _NS
    d_regs = d // _LANES
    zfill = max(z for z in range(8, _CHUNK + 1, 8) if rows_per_tile % z == 0)

    mesh = plsc.VectorSubcoreMesh(core_axis_name="c", subcore_axis_name="s",
                                  num_cores=_NC, num_subcores=_NS)

    @functools.partial(
        pl.kernel,
        out_type=jax.ShapeDtypeStruct((_NC, n_pad, d), jnp.float32),
        mesh=mesh,
        scratch_types=[
            pltpu.VMEM((_G, _CHUNK), jnp.int32),
            pltpu.VMEM((_G, _CHUNK), jnp.int32),
            pltpu.VMEM((_NBUF, _CHUNK, _LANES), jnp.float32),
            pltpu.VMEM((_NBUF, _CHUNK, d), jnp.float32),
            pltpu.VMEM_SHARED((n_pad, d), jnp.float32),
            pltpu.SemaphoreType.DMA((_NBUF,)),
            pltpu.SemaphoreType.DMA((_NBUF,)),
            pltpu.SemaphoreType.DMA((_NBUF,)),
        ],
    )
    def agg(x_hbm, src_hbm, dst_hbm, w_hbm, out_hbm,
            idx_s, idx_d, wv, rows, acc, gsem, wsem, ssem):
        c = lax.axis_index("c")
        s = lax.axis_index("s")
        wid = c * _NS + s
        base = wid * nch

        def zrow(r, carry):
            for k in range(d_regs):
                rows[0, r, pl.ds(k * _LANES, _LANES)] = jnp.zeros((_LANES,), jnp.float32)
            return carry
        lax.fori_loop(0, zfill, zrow, 0)
        for k in range(rows_per_tile // zfill):
            pltpu.sync_copy(
                rows.at[0, pl.ds(0, zfill)],
                acc.at[pl.ds(s * rows_per_tile + k * zfill, zfill)],
            )
        plsc.subcore_barrier()

        def issue_in(jl, j, b):
            pltpu.async_copy(x_hbm.at[idx_s.at[jl]], rows.at[b], gsem.at[b])
            pltpu.async_copy(w_hbm.at[base + j], wv.at[b], wsem.at[b])

        def wait_in(jl, j, b):
            pltpu.make_async_copy(x_hbm.at[idx_s.at[jl]], rows.at[b], gsem.at[b]).wait()
            pltpu.make_async_copy(w_hbm.at[base + j], wv.at[b], wsem.at[b]).wait()

        def scale(b):
            def row(r, rcarry):
                wvv = wv[b, r, :]
                for k in range(d_regs):
                    sl = pl.ds(k * _LANES, _LANES)
                    rows[b, r, sl] = rows[b, r, sl] * wvv
                return rcarry
            lax.fori_loop(0, _CHUNK, row, 0)

        def issue_scatter(jl, b):
            pltpu.async_copy(rows.at[b], acc.at[idx_d.at[jl]], ssem.at[b], add=True)

        def wait_scatter(jl, b):
            pltpu.make_async_copy(rows.at[b], acc.at[idx_d.at[jl]], ssem.at[b]).wait()

        for g in range(ngr):
            gbase = g * _G
            pltpu.sync_copy(src_hbm.at[pl.ds(base + gbase, _G)], idx_s)
            pltpu.sync_copy(dst_hbm.at[pl.ds(base + gbase, _G)], idx_d)
            issue_in(0, gbase + 0, 0)
            issue_in(1, gbase + 1, 1)

            def trio(t, carry, gbase=gbase):
                j0 = t * _NBUF
                for u in range(_NBUF):
                    jl = j0 + u
                    up = (u + 2) % _NBUF
                    wait_in(jl, gbase + jl, u)
                    scale(u)
                    issue_scatter(jl, u)
                    @pl.when(jl >= 1)
                    def _():
                        wait_scatter(jl - 1, up)

                    @pl.when(jl + 2 < _G)
                    def _():
                        issue_in(jl + 2, gbase + jl + 2, up)
                return carry
            lax.fori_loop(0, _G // _NBUF, trio, 0)
            wait_scatter(_G - 1, (_G - 1) % _NBUF)

        plsc.subcore_barrier()
        pltpu.sync_copy(
            acc.at[pl.ds(s * rows_per_tile, rows_per_tile)],
            out_hbm.at[c, pl.ds(s * rows_per_tile, rows_per_tile)],
        )

    return agg


def _mlp_bn_body(x_ref, p_ref, wa_ref, ba_ref, wb_ref, bb_ref, g_ref, be_ref, o_ref):
    n = x_ref.shape[0]
    h = x_ref[...] + p_ref[0, :n, :] + p_ref[1, :n, :]
    t = jnp.dot(h, wa_ref[...])
    t = jnp.maximum(t + ba_ref[...], 0.0)
    u = jnp.dot(t, wb_ref[...]) + bb_ref[...]
    u = jnp.maximum(u, 0.0)
    mu = jnp.mean(u, axis=0, keepdims=True)
    var = jnp.mean((u - mu) * (u - mu), axis=0, keepdims=True)
    o_ref[...] = (u - mu) * lax.rsqrt(var + _BN_EPS) * g_ref[...] + be_ref[...]


def _mlp_bn(x, p, wa, ba, wb, bb, g, be):
    n = x.shape[0]
    d_out = wb.shape[1]
    return pl.pallas_call(
        _mlp_bn_body,
        out_shape=jax.ShapeDtypeStruct((n, d_out), jnp.float32),
    )(x, p, wa, ba.reshape(1, -1), wb, bb.reshape(1, -1),
      g.reshape(1, -1), be.reshape(1, -1))


def kernel(x, edge_index, edge_attr, edge_weight,
           W1a, b1a, W1b, b1b, g1, be1,
           W2a, b2a, W2b, b2b, g2, be2):
    n_nodes, d = x.shape
    e = edge_index.shape[1]
    src = edge_index[0].astype(jnp.int32)
    dst = edge_index[1].astype(jnp.int32)
    w = edge_weight.astype(jnp.float32)

    grain = _NW * _CHUNK * _G
    e_pad = ((e + grain - 1) // grain) * grain
    nch = e_pad // (_NW * _CHUNK)
    pad = e_pad - e
    if pad:
        src = jnp.concatenate([src, jnp.zeros((pad,), jnp.int32)])
        dst = jnp.concatenate([dst, jnp.zeros((pad,), jnp.int32)])
        w = jnp.concatenate([w, jnp.zeros((pad,), jnp.float32)])
    src = src.reshape(_NW * nch, _CHUNK)
    dst = dst.reshape(_NW * nch, _CHUNK)
    w = jnp.broadcast_to(w[:, None], (e_pad, _LANES)).reshape(_NW * nch, _CHUNK, _LANES)

    ngrain = _NS * 128
    n_pad = ((n_nodes + ngrain - 1) // ngrain) * ngrain
    agg = _make_aggregate(n_pad, d, nch)

    p1 = agg(x, src, dst, w)
    h1 = _mlp_bn(x, p1, W1a, b1a, W1b, b1b, g1, be1)
    p2 = agg(h1, src, dst, w)
    return _mlp_bn(h1, p2, W2a, b2a, W2b, b2b, g2, be2)


# speedup vs baseline: 1.4671x; 1.4671x over previous
import functools

import jax
import jax.numpy as jnp
from jax import lax
from jax.experimental import pallas as pl
from jax.experimental.pallas import tpu as pltpu
from jax.experimental.pallas import tpu_sc as plsc

_BN_EPS = 1e-5
_NC = 2
_NS = 16
_NW = _NC * _NS
_CHUNK = 64
_LANES = 16
_G = 32


_NBUF = 2


@functools.lru_cache(maxsize=None)
def _make_aggregate(n_pad: int, d: int, nch0: int, nch1: int):
    assert d % _LANES == 0 and n_pad % (_NS * _CHUNK) == 0
    assert nch0 % _G == 0 and nch1 % _G == 0 and _G % _NBUF == 0
    ngr0 = nch0 // _G
    ngr1 = nch1 // _G
    rows_per_tile = n_pad // _NS
    d_regs = d // _LANES

    mesh = plsc.VectorSubcoreMesh(core_axis_name="c", subcore_axis_name="s",
                                  num_cores=_NC, num_subcores=_NS)

    @functools.partial(
        pl.kernel,
        out_type=jax.ShapeDtypeStruct((_NC, n_pad, d), jnp.float32),
        mesh=mesh,
        scratch_types=[
            pltpu.VMEM((_G, _CHUNK), jnp.int32),
            pltpu.VMEM((_G, _CHUNK), jnp.int32),
            pltpu.VMEM((_NBUF, _CHUNK, _LANES), jnp.float32),
            pltpu.VMEM((_NBUF, _CHUNK, d), jnp.float32),
            pltpu.VMEM_SHARED((n_pad, d), jnp.float32),
            pltpu.SemaphoreType.DMA((_NBUF,)),
            pltpu.SemaphoreType.DMA((_NBUF,)),
        ],
    )
    def agg(x_hbm, src_hbm, dst_hbm, w_hbm, out_hbm,
            idx_s, idx_d, wv, rows, acc, gsem, wsem):
        c = lax.axis_index("c")
        s = lax.axis_index("s")
        base = jnp.where(c == 0, s * nch0, _NS * nch0 + s * nch1)

        def zrow(r, carry):
            for k in range(d_regs):
                rows[0, r, pl.ds(k * _LANES, _LANES)] = jnp.zeros((_LANES,), jnp.float32)
            return carry
        lax.fori_loop(0, _CHUNK, zrow, 0)
        for k in range(rows_per_tile // _CHUNK):
            pltpu.sync_copy(
                rows.at[0],
                acc.at[pl.ds(s * rows_per_tile + k * _CHUNK, _CHUNK)],
            )
        plsc.subcore_barrier()

        def issue_in(jl, j, b):
            pltpu.async_copy(x_hbm.at[idx_s.at[jl]], rows.at[b], gsem.at[b])
            pltpu.async_copy(w_hbm.at[base + j], wv.at[b], wsem.at[b])

        def wait_in(jl, j, b):
            pltpu.make_async_copy(x_hbm.at[idx_s.at[jl]], rows.at[b], gsem.at[b]).wait()
            pltpu.make_async_copy(w_hbm.at[base + j], wv.at[b], wsem.at[b]).wait()

        def scale(b):
            def row(r, rcarry):
                wvv = wv[b, r, :]
                for k in range(d_regs):
                    sl = pl.ds(k * _LANES, _LANES)
                    rows[b, r, sl] = rows[b, r, sl] * wvv
                return rcarry
            lax.fori_loop(0, _CHUNK, row, 0)

        def run_group(g):
            gbase = g * _G
            pltpu.sync_copy(src_hbm.at[pl.ds(base + gbase, _G)], idx_s)
            pltpu.sync_copy(dst_hbm.at[pl.ds(base + gbase, _G)], idx_d)
            issue_in(0, gbase + 0, 0)
            issue_in(1, gbase + 1, 1)

            def pair(t, carry, gbase=gbase):
                j0 = t * _NBUF
                for u in range(_NBUF):
                    jl = j0 + u
                    wait_in(jl, gbase + jl, u)
                    scale(u)
                    pltpu.sync_copy(rows.at[u], acc.at[idx_d.at[jl]], add=True)
                    @pl.when(jl + 2 < _G)
                    def _():
                        issue_in(jl + 2, gbase + jl + 2, u)
                return carry
            lax.fori_loop(0, _G // _NBUF, pair, 0)

        big = 0 if ngr0 >= ngr1 else 1
        for g in range(min(ngr0, ngr1)):
            run_group(g)
        for g in range(min(ngr0, ngr1), max(ngr0, ngr1)):
            @pl.when(c == big)
            def _(g=g):
                run_group(g)

        plsc.subcore_barrier()
        pltpu.sync_copy(
            acc.at[pl.ds(s * rows_per_tile, rows_per_tile)],
            out_hbm.at[c, pl.ds(s * rows_per_tile, rows_per_tile)],
        )

    return agg


def _mlp_bn_body(x_ref, p_ref, wa_ref, ba_ref, wb_ref, bb_ref, g_ref, be_ref, o_ref):
    n = x_ref.shape[0]
    h = x_ref[...] + p_ref[0, :n, :] + p_ref[1, :n, :]
    t = jnp.dot(h, wa_ref[...])
    t = jnp.maximum(t + ba_ref[...], 0.0)
    u = jnp.dot(t, wb_ref[...]) + bb_ref[...]
    u = jnp.maximum(u, 0.0)
    mu = jnp.mean(u, axis=0, keepdims=True)
    var = jnp.mean((u - mu) * (u - mu), axis=0, keepdims=True)
    o_ref[...] = (u - mu) * lax.rsqrt(var + _BN_EPS) * g_ref[...] + be_ref[...]


def _mlp_bn(x, p, wa, ba, wb, bb, g, be):
    n = x.shape[0]
    d_out = wb.shape[1]
    return pl.pallas_call(
        _mlp_bn_body,
        out_shape=jax.ShapeDtypeStruct((n, d_out), jnp.float32),
    )(x, p, wa, ba.reshape(1, -1), wb, bb.reshape(1, -1),
      g.reshape(1, -1), be.reshape(1, -1))


def kernel(x, edge_index, edge_attr, edge_weight,
           W1a, b1a, W1b, b1b, g1, be1,
           W2a, b2a, W2b, b2b, g2, be2):
    n_nodes, d = x.shape
    e = edge_index.shape[1]
    src = edge_index[0].astype(jnp.int32)
    dst = edge_index[1].astype(jnp.int32)
    w = edge_weight.astype(jnp.float32)

    grain = _NS * _CHUNK * _G
    tot = ((e + grain - 1) // grain) * _G
    nch0 = max(_G, min(tot - _G, int(round(tot * 0.7 / _G)) * _G))
    nch1 = tot - nch0
    e_pad = _NS * tot * _CHUNK
    pad = e_pad - e
    if pad:
        src = jnp.concatenate([src, jnp.zeros((pad,), jnp.int32)])
        dst = jnp.concatenate([dst, jnp.zeros((pad,), jnp.int32)])
        w = jnp.concatenate([w, jnp.zeros((pad,), jnp.float32)])
    src = src.reshape(_NS * tot, _CHUNK)
    dst = dst.reshape(_NS * tot, _CHUNK)
    w = jnp.broadcast_to(w[:, None], (e_pad, _LANES)).reshape(_NS * tot, _CHUNK, _LANES)

    ngrain = _NS * _CHUNK
    n_pad = ((n_nodes + ngrain - 1) // ngrain) * ngrain
    agg = _make_aggregate(n_pad, d, nch0, nch1)

    p1 = agg(x, src, dst, w)
    h1 = _mlp_bn(x, p1, W1a, b1a, W1b, b1b, g1, be1)
    p2 = agg(h1, src, dst, w)
    return _mlp_bn(h1, p2, W2a, b2a, W2b, b2b, g2, be2)


# speedup vs baseline: 1.5103x; 1.0294x over previous
import functools

import jax
import jax.numpy as jnp
from jax import lax
from jax.experimental import pallas as pl
from jax.experimental.pallas import tpu as pltpu
from jax.experimental.pallas import tpu_sc as plsc

_BN_EPS = 1e-5
_NC = 2
_NS = 16
_NW = _NC * _NS
_CHUNK = 64
_LANES = 16
_G = 32


_NBUF = 2


@functools.lru_cache(maxsize=None)
def _make_aggregate(n_pad: int, d: int, nch0: int, nch1: int):
    assert d % _LANES == 0 and n_pad % (_NS * _CHUNK) == 0
    assert nch0 % _G == 0 and nch1 % _G == 0 and _G % _NBUF == 0
    ngr0 = nch0 // _G
    ngr1 = nch1 // _G
    rows_per_tile = n_pad // _NS
    d_regs = d // _LANES

    mesh = plsc.VectorSubcoreMesh(core_axis_name="c", subcore_axis_name="s",
                                  num_cores=_NC, num_subcores=_NS)

    @functools.partial(
        pl.kernel,
        out_type=jax.ShapeDtypeStruct((_NC, n_pad, d), jnp.float32),
        mesh=mesh,
        scratch_types=[
            pltpu.VMEM((_G, _CHUNK), jnp.int32),
            pltpu.VMEM((_G, _CHUNK), jnp.int32),
            pltpu.VMEM((_NBUF, _CHUNK, _LANES), jnp.float32),
            pltpu.VMEM((_NBUF, _CHUNK, d), jnp.float32),
            pltpu.VMEM_SHARED((n_pad, d), jnp.float32),
            pltpu.SemaphoreType.DMA((_NBUF,)),
            pltpu.SemaphoreType.DMA((_NBUF,)),
        ],
    )
    def agg(x_hbm, src_hbm, dst_hbm, w_hbm, out_hbm,
            idx_s, idx_d, wv, rows, acc, gsem, wsem):
        c = lax.axis_index("c")
        s = lax.axis_index("s")
        base = jnp.where(c == 0, s * nch0, _NS * nch0 + s * nch1)

        def zrow(r, carry):
            for k in range(d_regs):
                rows[0, r, pl.ds(k * _LANES, _LANES)] = jnp.zeros((_LANES,), jnp.float32)
            return carry
        lax.fori_loop(0, _CHUNK, zrow, 0)
        for k in range(rows_per_tile // _CHUNK):
            pltpu.sync_copy(
                rows.at[0],
                acc.at[pl.ds(s * rows_per_tile + k * _CHUNK, _CHUNK)],
            )
        plsc.subcore_barrier()

        def issue_in(jl, j, b):
            pltpu.async_copy(x_hbm.at[idx_s.at[jl]], rows.at[b], gsem.at[b])
            pltpu.async_copy(w_hbm.at[base + j], wv.at[b], wsem.at[b])

        def wait_in(jl, j, b):
            pltpu.make_async_copy(x_hbm.at[idx_s.at[jl]], rows.at[b], gsem.at[b]).wait()
            pltpu.make_async_copy(w_hbm.at[base + j], wv.at[b], wsem.at[b]).wait()

        def scale(b):
            def row(r, rcarry):
                wvv = wv[b, r, :]
                for k in range(d_regs):
                    sl = pl.ds(k * _LANES, _LANES)
                    rows[b, r, sl] = rows[b, r, sl] * wvv
                return rcarry
            lax.fori_loop(0, _CHUNK, row, 0)

        def run_group(g):
            gbase = g * _G
            pltpu.sync_copy(src_hbm.at[pl.ds(base + gbase, _G)], idx_s)
            pltpu.sync_copy(dst_hbm.at[pl.ds(base + gbase, _G)], idx_d)
            issue_in(0, gbase + 0, 0)
            issue_in(1, gbase + 1, 1)

            def pair(t, carry, gbase=gbase):
                j0 = t * _NBUF
                for u in range(_NBUF):
                    jl = j0 + u
                    wait_in(jl, gbase + jl, u)
                    scale(u)
                    pltpu.sync_copy(rows.at[u], acc.at[idx_d.at[jl]], add=True)
                    @pl.when(jl + 2 < _G)
                    def _():
                        issue_in(jl + 2, gbase + jl + 2, u)
                return carry
            lax.fori_loop(0, _G // _NBUF, pair, 0)

        big = 0 if ngr0 >= ngr1 else 1
        for g in range(min(ngr0, ngr1)):
            run_group(g)
        for g in range(min(ngr0, ngr1), max(ngr0, ngr1)):
            @pl.when(c == big)
            def _(g=g):
                run_group(g)

        plsc.subcore_barrier()
        pltpu.sync_copy(
            acc.at[pl.ds(s * rows_per_tile, rows_per_tile)],
            out_hbm.at[c, pl.ds(s * rows_per_tile, rows_per_tile)],
        )

    return agg


def _mlp_bn_body(x_ref, p_ref, wa_ref, ba_ref, wb_ref, bb_ref, g_ref, be_ref, o_ref):
    n = x_ref.shape[0]
    h = x_ref[...] + p_ref[0, :n, :] + p_ref[1, :n, :]
    t = jnp.dot(h, wa_ref[...])
    t = jnp.maximum(t + ba_ref[...], 0.0)
    u = jnp.dot(t, wb_ref[...]) + bb_ref[...]
    u = jnp.maximum(u, 0.0)
    mu = jnp.mean(u, axis=0, keepdims=True)
    var = jnp.mean((u - mu) * (u - mu), axis=0, keepdims=True)
    o_ref[...] = (u - mu) * lax.rsqrt(var + _BN_EPS) * g_ref[...] + be_ref[...]


def _mlp_bn(x, p, wa, ba, wb, bb, g, be):
    n = x.shape[0]
    d_out = wb.shape[1]
    return pl.pallas_call(
        _mlp_bn_body,
        out_shape=jax.ShapeDtypeStruct((n, d_out), jnp.float32),
    )(x, p, wa, ba.reshape(1, -1), wb, bb.reshape(1, -1),
      g.reshape(1, -1), be.reshape(1, -1))


def kernel(x, edge_index, edge_attr, edge_weight,
           W1a, b1a, W1b, b1b, g1, be1,
           W2a, b2a, W2b, b2b, g2, be2):
    n_nodes, d = x.shape
    e = edge_index.shape[1]
    src = edge_index[0].astype(jnp.int32)
    dst = edge_index[1].astype(jnp.int32)
    w = edge_weight.astype(jnp.float32)

    grain = _NS * _CHUNK * _G
    tot = ((e + grain - 1) // grain) * _G
    nch0 = max(_G, min(tot - _G, int(round(tot * 0.8 / _G)) * _G))
    nch1 = tot - nch0
    e_pad = _NS * tot * _CHUNK
    pad = e_pad - e
    if pad:
        src = jnp.concatenate([src, jnp.zeros((pad,), jnp.int32)])
        dst = jnp.concatenate([dst, jnp.zeros((pad,), jnp.int32)])
        w = jnp.concatenate([w, jnp.zeros((pad,), jnp.float32)])
    src = src.reshape(_NS * tot, _CHUNK)
    dst = dst.reshape(_NS * tot, _CHUNK)
    w = jnp.broadcast_to(w[:, None], (e_pad, _LANES)).reshape(_NS * tot, _CHUNK, _LANES)

    ngrain = _NS * _CHUNK
    n_pad = ((n_nodes + ngrain - 1) // ngrain) * ngrain
    agg = _make_aggregate(n_pad, d, nch0, nch1)

    p1 = agg(x, src, dst, w)
    h1 = _mlp_bn(x, p1, W1a, b1a, W1b, b1b, g1, be1)
    p2 = agg(h1, src, dst, w)
    return _mlp_bn(h1, p2, W2a, b2a, W2b, b2b, g2, be2)


# speedup vs baseline: 1.5981x; 1.0581x over previous
import functools

import jax
import jax.numpy as jnp
from jax import lax
from jax.experimental import pallas as pl
from jax.experimental.pallas import tpu as pltpu
from jax.experimental.pallas import tpu_sc as plsc

_BN_EPS = 1e-5
_NC = 2
_NS = 16
_NW = _NC * _NS
_CHUNK = 64
_LANES = 16
_G = 32


_NBUF = 2


@functools.lru_cache(maxsize=None)
def _make_aggregate(n_pad: int, d: int, nch0: int, nch1: int):
    assert d % _LANES == 0 and n_pad % (_NS * _CHUNK) == 0
    assert nch0 % _G == 0 and nch1 % _G == 0 and _G % _NBUF == 0
    ngr0 = nch0 // _G
    ngr1 = nch1 // _G
    rows_per_tile = n_pad // _NS
    d_regs = d // _LANES

    mesh = plsc.VectorSubcoreMesh(core_axis_name="c", subcore_axis_name="s",
                                  num_cores=_NC, num_subcores=_NS)

    @functools.partial(
        pl.kernel,
        out_type=jax.ShapeDtypeStruct((_NC, n_pad, d), jnp.float32),
        mesh=mesh,
        scratch_types=[
            pltpu.VMEM((_G, _CHUNK), jnp.int32),
            pltpu.VMEM((_G, _CHUNK), jnp.int32),
            pltpu.VMEM((_NBUF, _CHUNK, _LANES), jnp.float32),
            pltpu.VMEM((_NBUF, _CHUNK, d), jnp.float32),
            pltpu.VMEM_SHARED((n_pad, d), jnp.float32),
            pltpu.SemaphoreType.DMA((_NBUF,)),
            pltpu.SemaphoreType.DMA((_NBUF,)),
        ],
    )
    def agg(x_hbm, src_hbm, dst_hbm, w_hbm, out_hbm,
            idx_s, idx_d, wv, rows, acc, gsem, wsem):
        c = lax.axis_index("c")
        s = lax.axis_index("s")
        base = jnp.where(c == 0, s * nch0, _NS * nch0 + s * nch1)

        def zrow(r, carry):
            for k in range(d_regs):
                rows[0, r, pl.ds(k * _LANES, _LANES)] = jnp.zeros((_LANES,), jnp.float32)
            return carry
        lax.fori_loop(0, _CHUNK, zrow, 0)
        for k in range(rows_per_tile // _CHUNK):
            pltpu.sync_copy(
                rows.at[0],
                acc.at[pl.ds(s * rows_per_tile + k * _CHUNK, _CHUNK)],
            )
        plsc.subcore_barrier()

        def issue_in(jl, j, b):
            pltpu.async_copy(x_hbm.at[idx_s.at[jl]], rows.at[b], gsem.at[b])
            pltpu.async_copy(w_hbm.at[base + j], wv.at[b], wsem.at[b])

        def wait_in(jl, j, b):
            pltpu.make_async_copy(x_hbm.at[idx_s.at[jl]], rows.at[b], gsem.at[b]).wait()
            pltpu.make_async_copy(w_hbm.at[base + j], wv.at[b], wsem.at[b]).wait()

        def scale(b):
            def row(r, rcarry):
                wvv = wv[b, r, :]
                for k in range(d_regs):
                    sl = pl.ds(k * _LANES, _LANES)
                    rows[b, r, sl] = rows[b, r, sl] * wvv
                return rcarry
            lax.fori_loop(0, _CHUNK, row, 0)

        def run_group(g):
            gbase = g * _G
            pltpu.sync_copy(src_hbm.at[pl.ds(base + gbase, _G)], idx_s)
            pltpu.sync_copy(dst_hbm.at[pl.ds(base + gbase, _G)], idx_d)
            issue_in(0, gbase + 0, 0)
            issue_in(1, gbase + 1, 1)

            def pair(t, carry, gbase=gbase):
                j0 = t * _NBUF
                for u in range(_NBUF):
                    jl = j0 + u
                    wait_in(jl, gbase + jl, u)
                    scale(u)
                    pltpu.sync_copy(rows.at[u], acc.at[idx_d.at[jl]], add=True)
                    @pl.when(jl + 2 < _G)
                    def _():
                        issue_in(jl + 2, gbase + jl + 2, u)
                return carry
            lax.fori_loop(0, _G // _NBUF, pair, 0)

        big = 0 if ngr0 >= ngr1 else 1
        for g in range(min(ngr0, ngr1)):
            run_group(g)
        for g in range(min(ngr0, ngr1), max(ngr0, ngr1)):
            @pl.when(c == big)
            def _(g=g):
                run_group(g)

        plsc.subcore_barrier()
        pltpu.sync_copy(
            acc.at[pl.ds(s * rows_per_tile, rows_per_tile)],
            out_hbm.at[c, pl.ds(s * rows_per_tile, rows_per_tile)],
        )

    return agg


def _mlp_bn_body(x_ref, p_ref, wa_ref, ba_ref, wb_ref, bb_ref, g_ref, be_ref, o_ref):
    n = x_ref.shape[0]
    h = x_ref[...] + p_ref[0, :n, :] + p_ref[1, :n, :]
    t = jnp.dot(h, wa_ref[...])
    t = jnp.maximum(t + ba_ref[...], 0.0)
    u = jnp.dot(t, wb_ref[...]) + bb_ref[...]
    u = jnp.maximum(u, 0.0)
    mu = jnp.mean(u, axis=0, keepdims=True)
    var = jnp.mean((u - mu) * (u - mu), axis=0, keepdims=True)
    o_ref[...] = (u - mu) * lax.rsqrt(var + _BN_EPS) * g_ref[...] + be_ref[...]


def _mlp_bn(x, p, wa, ba, wb, bb, g, be):
    n = x.shape[0]
    d_out = wb.shape[1]
    return pl.pallas_call(
        _mlp_bn_body,
        out_shape=jax.ShapeDtypeStruct((n, d_out), jnp.float32),
    )(x, p, wa, ba.reshape(1, -1), wb, bb.reshape(1, -1),
      g.reshape(1, -1), be.reshape(1, -1))


def kernel(x, edge_index, edge_attr, edge_weight,
           W1a, b1a, W1b, b1b, g1, be1,
           W2a, b2a, W2b, b2b, g2, be2):
    n_nodes, d = x.shape
    e = edge_index.shape[1]
    src = edge_index[0].astype(jnp.int32)
    dst = edge_index[1].astype(jnp.int32)
    w = edge_weight.astype(jnp.float32)

    grain = _NS * _CHUNK * _G
    tot = ((e + grain - 1) // grain) * _G
    nch0 = max(_G, min(tot - _G, int(round(tot * 0.9 / _G)) * _G))
    nch1 = tot - nch0
    e_pad = _NS * tot * _CHUNK
    pad = e_pad - e
    if pad:
        src = jnp.concatenate([src, jnp.zeros((pad,), jnp.int32)])
        dst = jnp.concatenate([dst, jnp.zeros((pad,), jnp.int32)])
        w = jnp.concatenate([w, jnp.zeros((pad,), jnp.float32)])
    src = src.reshape(_NS * tot, _CHUNK)
    dst = dst.reshape(_NS * tot, _CHUNK)
    w = jnp.broadcast_to(w[:, None], (e_pad, _LANES)).reshape(_NS * tot, _CHUNK, _LANES)

    ngrain = _NS * _CHUNK
    n_pad = ((n_nodes + ngrain - 1) // ngrain) * ngrain
    agg = _make_aggregate(n_pad, d, nch0, nch1)

    p1 = agg(x, src, dst, w)
    h1 = _mlp_bn(x, p1, W1a, b1a, W1b, b1b, g1, be1)
    p2 = agg(h1, src, dst, w)
    return _mlp_bn(h1, p2, W2a, b2a, W2b, b2b, g2, be2)
